# baseline (speedup 1.0000x reference)
import functools

import jax
import jax.numpy as jnp
from jax import lax
from jax.experimental import pallas as pl
from jax.experimental.pallas import tpu as pltpu
from jax.experimental.pallas import tpu_sc as plsc

RATES = 5
HID = 64
HPAD = 128
NSUB = 16
NCHUNK = 16
CHUNK = 128
ACC_ROWS = 10240
NBUF = 2
ZROWS = 16


def _proj_body(feat_ref, w_ref, out_ref):
    res = lax.dot_general(
        feat_ref[0].astype(jnp.bfloat16), w_ref[...].astype(jnp.bfloat16),
        (((1,), (0,)), ((), ())), preferred_element_type=jnp.float32)
    out_ref[0] = jnp.concatenate([res, jnp.zeros_like(res)], axis=1)


def _project_rate(feat2, Wr):
    _, n, f = feat2.shape
    B = 2000
    return pl.pallas_call(
        _proj_body,
        grid=(2, n // B),
        in_specs=[
            pl.BlockSpec((1, B, f), lambda d, b: (d, b, 0)),
            pl.BlockSpec((f, HID), lambda d, b: (0, 0)),
        ],
        out_specs=pl.BlockSpec((1, B, HPAD), lambda d, b: (d, b, 0)),
        out_shape=jax.ShapeDtypeStruct((2, n, HPAD), jnp.float32),
    )(feat2, Wr)


def _segment_sums_rate(P_flat, src_idx, dst_idx, n, rate):
    mesh = plsc.VectorSubcoreMesh(core_axis_name="c", subcore_axis_name="s")
    rps = 624
    last_rows = n - rps * (NSUB - 1)

    @functools.partial(
        pl.kernel,
        mesh=mesh,
        out_type=jax.ShapeDtypeStruct((2 * n, HPAD), jnp.float32),
        scratch_types=[
            pltpu.VMEM((ZROWS, HPAD), jnp.float32),
            pltpu.VMEM((NCHUNK, CHUNK), jnp.int32),
            pltpu.VMEM((NCHUNK, CHUNK), jnp.int32),
            pltpu.VMEM((NBUF, CHUNK, HPAD), jnp.float32),
            pltpu.VMEM_SHARED((ACC_ROWS, HPAD), jnp.float32),
            pltpu.SemaphoreType.DMA,
            pltpu.SemaphoreType.DMA,
        ] + [pltpu.SemaphoreType.DMA] * NBUF,
    )
    def k(p_hbm, sidx_hbm, didx_hbm, out_hbm, zbuf, sidx_v, didx_v, gat_v,
          acc, isem0, isem1, *gsems):
        d = lax.axis_index("c")
        s = lax.axis_index("s")

        g = (d * RATES + rate) * NSUB + s
        ic0 = pltpu.async_copy(sidx_hbm.at[g], sidx_v, isem0)
        ic1 = pltpu.async_copy(didx_hbm.at[g], didx_v, isem1)

        @pl.loop(0, ZROWS)
        def _(i):
            for c in range(HPAD // 16):
                zbuf[i, pl.ds(c * 16, 16)] = jnp.zeros((16,), jnp.float32)

        @pl.when(s < NSUB - 1)
        def _():
            @pl.loop(0, rps // ZROWS)
            def _(kk):
                pltpu.sync_copy(
                    zbuf, acc.at[pl.ds(s * rps + kk * ZROWS, ZROWS)])

        @pl.when(s == NSUB - 1)
        def _():
            @pl.loop(0, (n - (NSUB - 1) * rps) // ZROWS)
            def _(kk):
                pltpu.sync_copy(
                    zbuf, acc.at[pl.ds((NSUB - 1) * rps + kk * ZROWS, ZROWS)])

        ic0.wait()
        ic1.wait()
        plsc.subcore_barrier()

        gathers = [
            pltpu.async_copy(p_hbm.at[sidx_v.at[b]], gat_v.at[b], gsems[b])
            for b in range(NBUF)
        ]
        for j in range(NCHUNK):
            b = j % NBUF
            gathers[b].wait()
            pltpu.sync_copy(gat_v.at[b], acc.at[didx_v.at[j]], add=True)
            if j + NBUF < NCHUNK:
                gathers[b] = pltpu.async_copy(
                    p_hbm.at[sidx_v.at[j + NBUF]], gat_v.at[b], gsems[b])
        plsc.subcore_barrier()

        @pl.when(s < NSUB - 1)
        def _():
            pltpu.sync_copy(
                acc.at[pl.ds(s * rps, rps)],
                out_hbm.at[pl.ds(d * n + s * rps, rps)])

        @pl.when(s == NSUB - 1)
        def _():
            pltpu.sync_copy(
                acc.at[pl.ds((NSUB - 1) * rps, last_rows)],
                out_hbm.at[pl.ds(d * n + (NSUB - 1) * rps, last_rows)])

    return k(P_flat, src_idx, dst_idx)


def _bf16_dot(a, b):
    return lax.dot_general(a.astype(jnp.bfloat16), b.astype(jnp.bfloat16),
                           (((1,), (0,)), ((), ())),
                           preferred_element_type=jnp.float32)


def _embed_body(h0, h1, h2, h3, h4, side_ref, wside_ref, bias_ref, wcat_ref,
                out_u_ref, out_v_ref):
    hs = (h0, h1, h2, h3, h4)
    for d, out_ref in ((0, out_u_ref), (1, out_v_ref)):
        side = _bf16_dot(side_ref[d], wside_ref[...])
        side = jnp.maximum(side + bias_ref[d], 0.0)
        acc = _bf16_dot(side, wcat_ref[d, RATES * HPAD:, :])
        for r in range(RATES):
            acc += _bf16_dot(jnp.maximum(hs[r][d], 0.0),
                             wcat_ref[d, r * HPAD:(r + 1) * HPAD, :])
        out_ref[...] = jnp.maximum(acc, 0.0)


def _embed(Hs, side2, W_side, bias2, wcat2):
    _, n, _ = Hs[0].shape
    sdim = side2.shape[2]
    out_dim = wcat2.shape[2]
    B = 2000
    return pl.pallas_call(
        _embed_body,
        grid=(n // B,),
        in_specs=[
            pl.BlockSpec((2, B, HPAD), lambda b: (0, b, 0))
            for _ in range(RATES)
        ] + [
            pl.BlockSpec((2, B, sdim), lambda b: (0, b, 0)),
            pl.BlockSpec((sdim, wcat2.shape[1] - RATES * HPAD),
                         lambda b: (0, 0)),
            pl.BlockSpec((2, 1, bias2.shape[2]), lambda b: (0, 0, 0)),
            pl.BlockSpec((2, wcat2.shape[1], out_dim), lambda b: (0, 0, 0)),
        ],
        out_specs=[pl.BlockSpec((B, out_dim), lambda b: (b, 0)),
                   pl.BlockSpec((B, out_dim), lambda b: (b, 0))],
        out_shape=[jax.ShapeDtypeStruct((n, out_dim), jnp.float32),
                   jax.ShapeDtypeStruct((n, out_dim), jnp.float32)],
    )(*Hs, side2, W_side, bias2, wcat2)


def kernel(feature_u, feature_v, edge_rows, edge_cols, side_feature_u,
           side_feature_v, W, W_side, bias_u, bias_v, W_cat_u, W_cat_v):
    n = feature_u.shape[0]
    rows = edge_rows.astype(jnp.int32)
    cols = edge_cols.astype(jnp.int32)

    feat2 = jnp.stack([feature_u, feature_v])

    src = jnp.stack([cols + n, rows])
    dst = jnp.stack([rows, cols])
    e = src.shape[2]
    pad = NSUB * NCHUNK * CHUNK - e
    src = jnp.concatenate(
        [src, jnp.zeros((2, RATES, pad), jnp.int32)], axis=2)
    dst = jnp.concatenate(
        [dst, jnp.full((2, RATES, pad), n, jnp.int32)], axis=2)
    src = src.reshape(2 * RATES * NSUB, NCHUNK, CHUNK)
    dst = dst.reshape(2 * RATES * NSUB, NCHUNK, CHUNK)

    Hs = []
    for r in range(RATES):
        P_r = _project_rate(feat2, W[r])
        H_r = _segment_sums_rate(
            P_r.reshape(2 * n, HPAD), src, dst, n, r)
        Hs.append(H_r.reshape(2, n, HPAD))

    side2 = jnp.stack([side_feature_u, side_feature_v])
    bias2 = jnp.stack([bias_u, bias_v])[:, None, :]
    wcat2 = jnp.stack([W_cat_u, W_cat_v])
    wcat_rates = wcat2[:, :RATES * HID, :].reshape(2, RATES, HID, -1)
    wcat_rates = jnp.concatenate(
        [wcat_rates,
         jnp.zeros((2, RATES, HPAD - HID, wcat2.shape[2]), jnp.float32)],
        axis=2).reshape(2, RATES * HPAD, -1)
    wcat2 = jnp.concatenate([wcat_rates, wcat2[:, RATES * HID:, :]], axis=1)

    emb_u, emb_v = _embed(Hs, side2, W_side, bias2, wcat2)
    return emb_u, emb_v


# speedup vs baseline: 1.0419x; 1.0419x over previous
import functools

import jax
import jax.numpy as jnp
from jax import lax
from jax.experimental import pallas as pl
from jax.experimental.pallas import tpu as pltpu
from jax.experimental.pallas import tpu_sc as plsc

RATES = 5
HID = 64
HPAD = 128
NSUB = 16
NCHUNK = 32
CHUNK = 64
ACC_ROWS = 10240
NBUF = 4
ZROWS = 16


def _proj_body(feat_ref, w_ref, out_ref):
    res = lax.dot_general(
        feat_ref[0].astype(jnp.bfloat16), w_ref[0].astype(jnp.bfloat16),
        (((1,), (0,)), ((), ())), preferred_element_type=jnp.float32)
    out_ref[0, 0] = jnp.concatenate(
        [res, jnp.zeros_like(res)], axis=1)


def _project(feat2, W):
    _, n, f = feat2.shape
    B = 2000
    return pl.pallas_call(
        _proj_body,
        grid=(2, n // B, RATES),
        in_specs=[
            pl.BlockSpec((1, B, f), lambda d, b, r: (d, b, 0)),
            pl.BlockSpec((1, f, HID), lambda d, b, r: (r, 0, 0)),
        ],
        out_specs=pl.BlockSpec((1, 1, B, HPAD), lambda d, b, r: (d, r, b, 0)),
        out_shape=jax.ShapeDtypeStruct((2, RATES, n, HPAD), jnp.float32),
    )(feat2, W)


def _segment_sums(P_flat, src_idx, dst_idx, n):
    mesh = plsc.VectorSubcoreMesh(core_axis_name="c", subcore_axis_name="s")
    rps = 624
    last_rows = n - rps * (NSUB - 1)

    @functools.partial(
        pl.kernel,
        mesh=mesh,
        out_type=jax.ShapeDtypeStruct((2 * RATES * n, HPAD), jnp.float32),
        scratch_types=[
            pltpu.VMEM((ZROWS, HPAD), jnp.float32),
            pltpu.VMEM((NCHUNK, CHUNK), jnp.int32),
            pltpu.VMEM((NCHUNK, CHUNK), jnp.int32),
            pltpu.VMEM((NBUF, CHUNK, HPAD), jnp.float32),
            pltpu.VMEM_SHARED((ACC_ROWS, HPAD), jnp.float32),
            pltpu.SemaphoreType.DMA,
            pltpu.SemaphoreType.DMA,
        ] + [pltpu.SemaphoreType.DMA] * NBUF,
    )
    def k(p_hbm, sidx_hbm, didx_hbm, out_hbm, zbuf, sidx_v, didx_v, gat_v,
          acc, isem0, isem1, *gsems):
        d = lax.axis_index("c")
        s = lax.axis_index("s")

        @pl.loop(0, ZROWS)
        def _(i):
            for c in range(HPAD // 16):
                zbuf[i, pl.ds(c * 16, 16)] = jnp.zeros((16,), jnp.float32)

        def zero_stripe():
            @pl.when(s < NSUB - 1)
            def _():
                @pl.loop(0, rps // ZROWS)
                def _(kk):
                    pltpu.sync_copy(
                        zbuf, acc.at[pl.ds(s * rps + kk * ZROWS, ZROWS)])

            @pl.when(s == NSUB - 1)
            def _():
                @pl.loop(0, (n - (NSUB - 1) * rps) // ZROWS)
                def _(kk):
                    pltpu.sync_copy(
                        zbuf,
                        acc.at[pl.ds((NSUB - 1) * rps + kk * ZROWS, ZROWS)])

        zero_stripe()

        @pl.loop(0, RATES)
        def _(r):
            g = (d * RATES + r) * NSUB + s
            ic0 = pltpu.async_copy(sidx_hbm.at[g], sidx_v, isem0)
            ic1 = pltpu.async_copy(didx_hbm.at[g], didx_v, isem1)
            plsc.subcore_barrier()

            ic0.wait()
            gathers = [
                pltpu.async_copy(p_hbm.at[sidx_v.at[b]], gat_v.at[b],
                                 gsems[b])
                for b in range(NBUF)
            ]
            ic1.wait()
            for j in range(NCHUNK):
                b = j % NBUF
                gathers[b].wait()
                pltpu.sync_copy(gat_v.at[b], acc.at[didx_v.at[j]], add=True)
                if j + NBUF < NCHUNK:
                    gathers[b] = pltpu.async_copy(
                        p_hbm.at[sidx_v.at[j + NBUF]], gat_v.at[b], gsems[b])
            plsc.subcore_barrier()

            @pl.when(s < NSUB - 1)
            def _():
                pltpu.sync_copy(
                    acc.at[pl.ds(s * rps, rps)],
                    out_hbm.at[pl.ds((d * RATES + r) * n + s * rps, rps)])

            @pl.when(s == NSUB - 1)
            def _():
                pltpu.sync_copy(
                    acc.at[pl.ds((NSUB - 1) * rps, last_rows)],
                    out_hbm.at[pl.ds((d * RATES + r) * n + (NSUB - 1) * rps,
                                     last_rows)])

            @pl.when(r < RATES - 1)
            def _():
                zero_stripe()

    return k(P_flat, src_idx, dst_idx)


def _bf16_dot(a, b):
    return lax.dot_general(a.astype(jnp.bfloat16), b.astype(jnp.bfloat16),
                           (((1,), (0,)), ((), ())),
                           preferred_element_type=jnp.float32)


def _embed_body(h_ref, side_ref, wside_ref, bias_ref, wcat_ref,
                out_u_ref, out_v_ref):
    for d, out_ref in ((0, out_u_ref), (1, out_v_ref)):
        side = _bf16_dot(side_ref[d], wside_ref[...])
        side = jnp.maximum(side + bias_ref[d], 0.0)
        acc = _bf16_dot(side, wcat_ref[d, RATES * HPAD:, :])
        for r in range(RATES):
            acc += _bf16_dot(jnp.maximum(h_ref[d, r], 0.0),
                             wcat_ref[d, r * HPAD:(r + 1) * HPAD, :])
        out_ref[...] = jnp.maximum(acc, 0.0)


def _embed(H, side2, W_side, bias2, wcat2):
    _, _, n, _ = H.shape
    sdim = side2.shape[2]
    out_dim = wcat2.shape[2]
    B = 2000
    return pl.pallas_call(
        _embed_body,
        grid=(n // B,),
        in_specs=[
            pl.BlockSpec((2, RATES, B, HPAD), lambda b: (0, 0, b, 0)),
            pl.BlockSpec((2, B, sdim), lambda b: (0, b, 0)),
            pl.BlockSpec((sdim, wcat2.shape[1] - RATES * HPAD),
                         lambda b: (0, 0)),
            pl.BlockSpec((2, 1, bias2.shape[2]), lambda b: (0, 0, 0)),
            pl.BlockSpec((2, wcat2.shape[1], out_dim), lambda b: (0, 0, 0)),
        ],
        out_specs=[pl.BlockSpec((B, out_dim), lambda b: (b, 0)),
                   pl.BlockSpec((B, out_dim), lambda b: (b, 0))],
        out_shape=[jax.ShapeDtypeStruct((n, out_dim), jnp.float32),
                   jax.ShapeDtypeStruct((n, out_dim), jnp.float32)],
    )(H, side2, W_side, bias2, wcat2)


def kernel(feature_u, feature_v, edge_rows, edge_cols, side_feature_u,
           side_feature_v, W, W_side, bias_u, bias_v, W_cat_u, W_cat_v):
    n = feature_u.shape[0]
    rows = edge_rows.astype(jnp.int32)
    cols = edge_cols.astype(jnp.int32)

    feat2 = jnp.stack([feature_u, feature_v])
    P = _project(feat2, W)
    P_flat = P.reshape(2 * RATES * n, HPAD)

    roff = (jnp.arange(RATES, dtype=jnp.int32) * n)[:, None]
    src = jnp.stack([cols + roff + RATES * n, rows + roff])
    dst = jnp.stack([rows, cols])
    e = src.shape[2]
    pad = NSUB * NCHUNK * CHUNK - e
    src = jnp.concatenate(
        [src, jnp.zeros((2, RATES, pad), jnp.int32)], axis=2)
    dst = jnp.concatenate(
        [dst, jnp.full((2, RATES, pad), n, jnp.int32)], axis=2)
    src = src.reshape(2 * RATES * NSUB, NCHUNK, CHUNK)
    dst = dst.reshape(2 * RATES * NSUB, NCHUNK, CHUNK)

    H = _segment_sums(P_flat, src, dst, n).reshape(2, RATES, n, HPAD)

    side2 = jnp.stack([side_feature_u, side_feature_v])
    bias2 = jnp.stack([bias_u, bias_v])[:, None, :]
    wcat2 = jnp.stack([W_cat_u, W_cat_v])
    wcat_rates = wcat2[:, :RATES * HID, :].reshape(2, RATES, HID, -1)
    wcat_rates = jnp.concatenate(
        [wcat_rates,
         jnp.zeros((2, RATES, HPAD - HID, wcat2.shape[2]), jnp.float32)],
        axis=2).reshape(2, RATES * HPAD, -1)
    wcat2 = jnp.concatenate([wcat_rates, wcat2[:, RATES * HID:, :]], axis=1)
    emb_u, emb_v = _embed(H, side2, W_side, bias2, wcat2)
    return emb_u, emb_v
